# baseline (speedup 1.0000x reference)
import jax
import jax.numpy as jnp
from jax.experimental import pallas as pl

N = 10000
E = 320000
D = 128
H1, C1 = 8, 16
H2, C2 = 1, 32


def _gat_layer(x, src, dst, W, a_src, a_dst, b, heads, ch, concat):
    h = (x @ W).reshape(N, heads, ch)
    alpha_src = jnp.sum(h * a_src, axis=-1)
    alpha_dst = jnp.sum(h * a_dst, axis=-1)
    alpha = alpha_src[src] + alpha_dst[dst]
    alpha = jax.nn.leaky_relu(alpha, negative_slope=0.2)
    amax = jax.ops.segment_max(alpha, dst, num_segments=N)
    ealpha = jnp.exp(alpha - amax[dst])
    denom = jax.ops.segment_sum(ealpha, dst, num_segments=N)
    coef = ealpha / (denom[dst] + 1e-16)
    out = jax.ops.segment_sum(h[src] * coef[..., None], dst, num_segments=N)
    if concat:
        out = out.reshape(N, heads * ch)
    else:
        out = jnp.mean(out, axis=1)
    return out + b


def _identity_kernel(x_ref, o_ref):
    o_ref[...] = x_ref[...]


def kernel(x, edge_index, W1, att_src1, att_dst1, b1, W2, att_src2, att_dst2, b2, Wh, bh):
    loop = jnp.arange(N, dtype=edge_index.dtype)
    src = jnp.concatenate([edge_index[0], loop])
    dst = jnp.concatenate([edge_index[1], loop])
    x1 = jax.nn.elu(_gat_layer(x, src, dst, W1, att_src1, att_dst1, b1, H1, C1, True))
    x2 = jax.nn.elu(_gat_layer(x1, src, dst, W2, att_src2, att_dst2, b2, H2, C2, False))
    out = x2 @ Wh + bh
    return pl.pallas_call(
        _identity_kernel,
        out_shape=jax.ShapeDtypeStruct(out.shape, out.dtype),
    )(out)


# speedup vs baseline: 47.9746x; 47.9746x over previous
import functools

import jax
import jax.numpy as jnp
from jax import lax
from jax.experimental import pallas as pl
from jax.experimental.pallas import tpu as pltpu
from jax.experimental.pallas import tpu_sc as plsc

N = 10000
E = 320000
D = 128
H1, C1 = 8, 16
H2, C2 = 1, 32

NROWS = 10016
B = 128
NC, NS = 2, 16
NW = NC * NS
EP = E + N
CH = -(-EP // (NW * B))
E_PAD = NW * CH * B
RPT = NROWS // NS


def _edge_pass(HC, H, C, n_f_rows, n_g_rows):
    HCp = HC + 16
    mesh = plsc.VectorSubcoreMesh(
        core_axis_name="c", subcore_axis_name="s", num_cores=NC, num_subcores=NS)

    def body(F_hbm, G_hbm, srcm_hbm, dstm_hbm, maxs_hbm, out_hbm,
             src_v, dst_v, Fs_v, Gd_v, buf_v, maxs_v, accum, sem1, sem2):
        c = lax.axis_index("c")
        s = lax.axis_index("s")
        wid = s * NC + c
        row0 = s * RPT

        zero16 = jnp.zeros((16,), jnp.float32)

        def zrow(r, carry):
            for k in range(HCp // 16):
                buf_v[r, pl.ds(k * 16, 16)] = zero16
            return carry

        lax.fori_loop(0, B, zrow, 0)
        off = 0
        while off < RPT:
            step = min(B, RPT - off)
            pltpu.sync_copy(buf_v.at[pl.ds(0, step)],
                            accum.at[pl.ds(row0 + off, step)])
            off += step

        pltpu.sync_copy(maxs_hbm, maxs_v)
        maxs = maxs_v[...]
        lane = lax.iota(jnp.int32, 16)
        hmask = lane < H

        plsc.subcore_barrier()

        def chunk(j, carry):
            pltpu.sync_copy(srcm_hbm.at[wid, j], src_v)
            pltpu.sync_copy(dstm_hbm.at[wid, j], dst_v)
            cps = pltpu.async_copy(F_hbm.at[src_v], Fs_v, sem1)
            cpd = pltpu.async_copy(G_hbm.at[dst_v], Gd_v, sem2)
            cps.wait()
            cpd.wait()

            def edge(e, ecarry):
                As = Fs_v[e, pl.ds(HC, 16)]
                Gd = Gd_v[e, :]
                a1 = As + Gd
                a2 = maxs + Gd
                s1 = jnp.where(a1 >= 0, a1, 0.2 * a1)
                s2 = jnp.where(a2 >= 0, a2, 0.2 * a2)
                w = jnp.where(hmask, jnp.exp(s1 - s2), 0.0)
                buf_v[e, pl.ds(HC, 16)] = w
                dnums = lax.GatherDimensionNumbers(
                    offset_dims=(), collapsed_slice_dims=(0,), start_index_map=(0,))
                for k in range(HC // 16):
                    head = (k * 16) // C
                    idx = jnp.full((16, 1), head, jnp.int32)
                    mult = lax.gather(w, idx, dnums, (1,),
                                      mode=lax.GatherScatterMode.PROMISE_IN_BOUNDS)
                    buf_v[e, pl.ds(k * 16, 16)] = Fs_v[e, pl.ds(k * 16, 16)] * mult
                return ecarry

            lax.fori_loop(0, B, edge, 0)
            pltpu.sync_copy(buf_v, accum.at[dst_v], add=True)
            return carry

        lax.fori_loop(0, CH, chunk, 0)
        plsc.subcore_barrier()
        pltpu.sync_copy(accum.at[pl.ds(row0, RPT)], out_hbm.at[c, pl.ds(row0, RPT)])

    return pl.kernel(
        body,
        out_type=jax.ShapeDtypeStruct((NC, NROWS, HCp), jnp.float32),
        mesh=mesh,
        scratch_types=[
            pltpu.VMEM((B,), jnp.int32),
            pltpu.VMEM((B,), jnp.int32),
            pltpu.VMEM((B, HCp), jnp.float32),
            pltpu.VMEM((B, 16), jnp.float32),
            pltpu.VMEM((B, HCp), jnp.float32),
            pltpu.VMEM((16,), jnp.float32),
            pltpu.VMEM_SHARED((NROWS, HCp), jnp.float32),
            pltpu.SemaphoreType.DMA,
            pltpu.SemaphoreType.DMA,
        ],
        compiler_params=pltpu.CompilerParams(use_tc_tiling_on_sc=False),
    )


def _tc1_body(x_ref, W_ref, Am_ref, Ad_ref, F_ref, G_ref, mx_ref):
    h = jnp.dot(x_ref[...], W_ref[...], preferred_element_type=jnp.float32)
    asrc = jnp.dot(h, Am_ref[...], preferred_element_type=jnp.float32)
    adst = jnp.dot(h, Ad_ref[...], preferred_element_type=jnp.float32)
    z = jnp.zeros_like(asrc)
    F_ref[:, :D] = h
    F_ref[:, D:D + 8] = asrc
    F_ref[:, D + 8:] = z
    G_ref[:, :8] = adst
    G_ref[:, 8:] = z
    bm = jnp.max(asrc, axis=0, keepdims=True)
    bm16 = jnp.concatenate([bm, jnp.zeros((1, 8), jnp.float32)], axis=1)
    pi = pl.program_id(0)

    @pl.when(pi == 0)
    def _():
        mx_ref[...] = bm16

    @pl.when(pi != 0)
    def _():
        mx_ref[...] = jnp.maximum(mx_ref[...], bm16)


def _elu(v):
    return jnp.where(v > 0, v, jnp.exp(jnp.minimum(v, 0.0)) - 1.0)


def _tc2_body(P_ref, b1_ref, W2_ref, S_ref, Am_ref, Ad_ref, F_ref, G_ref, mx_ref):
    p0 = P_ref[0]
    p1 = P_ref[1]
    num = p0[:, :D] + p1[:, :D]
    den8 = p0[:, D:D + 8] + p1[:, D:D + 8]
    den = jnp.dot(den8, S_ref[...], preferred_element_type=jnp.float32)
    x1 = _elu(num / (den + 1e-16) + b1_ref[...])
    h2 = jnp.dot(x1, W2_ref[...], preferred_element_type=jnp.float32)
    asrc16 = jnp.dot(h2, Am_ref[...], preferred_element_type=jnp.float32)
    adst16 = jnp.dot(h2, Ad_ref[...], preferred_element_type=jnp.float32)
    F_ref[:, :32] = h2
    F_ref[:, 32:] = asrc16
    G_ref[...] = adst16
    pi = pl.program_id(0)
    bm16 = jnp.max(asrc16, axis=0, keepdims=True)

    @pl.when(pi == 0)
    def _():
        mx_ref[...] = bm16

    @pl.when(pi != 0)
    def _():
        mx_ref[...] = jnp.maximum(mx_ref[...], bm16)


def _tc3_body(P_ref, b2_ref, Wh_ref, bh_ref, o_ref):
    p0 = P_ref[0]
    p1 = P_ref[1]
    num = p0[:, :32] + p1[:, :32]
    den = p0[:, 32:33] + p1[:, 32:33]
    x2 = _elu(num / (den + 1e-16) + b2_ref[...])
    o_ref[...] = jnp.dot(x2, Wh_ref[...], preferred_element_type=jnp.float32) + bh_ref[...]


def kernel(x, edge_index, W1, att_src1, att_dst1, b1, W2, att_src2, att_dst2, b2, Wh, bh):
    f32 = jnp.float32
    eye8 = jnp.eye(8, dtype=f32)
    Am1 = jnp.kron(eye8, jnp.ones((C1, 1), f32)) * att_src1.reshape(D, 1)
    Ad1 = jnp.kron(eye8, jnp.ones((C1, 1), f32)) * att_dst1.reshape(D, 1)
    S = jnp.kron(eye8, jnp.ones((1, C1), f32))
    Am2 = jnp.concatenate([att_src2.reshape(C2, 1), jnp.zeros((C2, 15), f32)], axis=1)
    Ad2 = jnp.concatenate([att_dst2.reshape(C2, 1), jnp.zeros((C2, 15), f32)], axis=1)

    loop = jnp.arange(N, dtype=jnp.int32)
    pad = E_PAD - EP
    srcm = jnp.concatenate([edge_index[0], loop, jnp.zeros((pad,), jnp.int32)]).reshape(NW, CH, B)
    dstm = jnp.concatenate([edge_index[1], loop, jnp.full((pad,), N, jnp.int32)]).reshape(NW, CH, B)

    F1, G1, mx1 = pl.pallas_call(
        _tc1_body,
        grid=(10,),
        in_specs=[
            pl.BlockSpec((1000, D), lambda i: (i, 0)),
            pl.BlockSpec((D, D), lambda i: (0, 0)),
            pl.BlockSpec((D, 8), lambda i: (0, 0)),
            pl.BlockSpec((D, 8), lambda i: (0, 0)),
        ],
        out_specs=[
            pl.BlockSpec((1000, D + 16), lambda i: (i, 0)),
            pl.BlockSpec((1000, 16), lambda i: (i, 0)),
            pl.BlockSpec((1, 16), lambda i: (0, 0)),
        ],
        out_shape=[
            jax.ShapeDtypeStruct((N, D + 16), f32),
            jax.ShapeDtypeStruct((N, 16), f32),
            jax.ShapeDtypeStruct((1, 16), f32),
        ],
    )(x, W1, Am1, Ad1)

    G1p = jnp.concatenate([G1, jnp.zeros((1, 16), f32)], axis=0)

    part1 = _edge_pass(D, H1, C1, N, N + 1)(F1, G1p, srcm, dstm, mx1.reshape(16))

    F2, G2, mx2 = pl.pallas_call(
        _tc2_body,
        grid=(10,),
        in_specs=[
            pl.BlockSpec((2, 1000, D + 16), lambda i: (0, i, 0)),
            pl.BlockSpec((1, D), lambda i: (0, 0)),
            pl.BlockSpec((D, 32), lambda i: (0, 0)),
            pl.BlockSpec((8, D), lambda i: (0, 0)),
            pl.BlockSpec((32, 16), lambda i: (0, 0)),
            pl.BlockSpec((32, 16), lambda i: (0, 0)),
        ],
        out_specs=[
            pl.BlockSpec((1000, 48), lambda i: (i, 0)),
            pl.BlockSpec((1000, 16), lambda i: (i, 0)),
            pl.BlockSpec((1, 16), lambda i: (0, 0)),
        ],
        out_shape=[
            jax.ShapeDtypeStruct((N, 48), f32),
            jax.ShapeDtypeStruct((N, 16), f32),
            jax.ShapeDtypeStruct((1, 16), f32),
        ],
    )(part1[:, :N], b1.reshape(1, D), W2, S, Am2, Ad2)

    G2p = jnp.concatenate([G2, jnp.zeros((1, 16), f32)], axis=0)

    part2 = _edge_pass(32, H2, C2, N, N + 1)(F2, G2p, srcm, dstm, mx2.reshape(16))

    out = pl.pallas_call(
        _tc3_body,
        grid=(10,),
        in_specs=[
            pl.BlockSpec((2, 1000, 48), lambda i: (0, i, 0)),
            pl.BlockSpec((1, 32), lambda i: (0, 0)),
            pl.BlockSpec((32, 1), lambda i: (0, 0)),
            pl.BlockSpec((1, 1), lambda i: (0, 0)),
        ],
        out_specs=pl.BlockSpec((1000, 1), lambda i: (i, 0)),
        out_shape=jax.ShapeDtypeStruct((N, 1), f32),
    )(part2[:, :N], b2.reshape(1, 32), Wh, bh.reshape(1, 1))

    return out


# speedup vs baseline: 85.7922x; 1.7883x over previous
import functools

import jax
import jax.numpy as jnp
from jax import lax
from jax.experimental import pallas as pl
from jax.experimental.pallas import tpu as pltpu
from jax.experimental.pallas import tpu_sc as plsc

N = 10000
E = 320000
D = 128
H1, C1 = 8, 16
H2, C2 = 1, 32

NROWS = 10016
B = 128
NC, NS = 2, 16
NW = NC * NS
EP = E + N
CH = -(-EP // (NW * B))
E_PAD = NW * CH * B
RPT = NROWS // NS


def _edge_pass(HC, H, C, n_f_rows, n_g_rows):
    HCp = HC + 16
    mesh = plsc.VectorSubcoreMesh(
        core_axis_name="c", subcore_axis_name="s", num_cores=NC, num_subcores=NS)

    def body(F_hbm, G_hbm, srcm_hbm, dstm_hbm, maxs_hbm, out_hbm,
             src0, dst0, src1, dst1, Fs_v, Gd_v, buf_v, maxs_v, accum,
             semF, semG, semI, semS):
        c = lax.axis_index("c")
        s = lax.axis_index("s")
        wid = s * NC + c
        row0 = s * RPT

        zero16 = jnp.zeros((16,), jnp.float32)

        def zrow(r, carry):
            for k in range(HCp // 16):
                buf_v[r, pl.ds(k * 16, 16)] = zero16
            return carry

        lax.fori_loop(0, B, zrow, 0)
        off = 0
        while off < RPT:
            step = min(B, RPT - off)
            pltpu.sync_copy(buf_v.at[pl.ds(0, step)],
                            accum.at[pl.ds(row0 + off, step)])
            off += step

        pltpu.sync_copy(maxs_hbm, maxs_v)
        maxs = maxs_v[...]
        lane = lax.iota(jnp.int32, 16)
        hmask = lane < H
        dnums = lax.GatherDimensionNumbers(
            offset_dims=(), collapsed_slice_dims=(0,), start_index_map=(0,))

        plsc.subcore_barrier()

        pltpu.sync_copy(srcm_hbm.at[wid, 0], src0)
        pltpu.sync_copy(dstm_hbm.at[wid, 0], dst0)
        pltpu.async_copy(F_hbm.at[src0], Fs_v, semF)
        pltpu.async_copy(G_hbm.at[dst0], Gd_v, semG)

        def step(j, sa, da, sb, db):
            pltpu.make_async_copy(F_hbm.at[sa], Fs_v, semF).wait()
            pltpu.make_async_copy(G_hbm.at[da], Gd_v, semG).wait()

            @pl.when(j + 1 < CH)
            def _():
                pltpu.async_copy(srcm_hbm.at[wid, j + 1], sb, semI)
                pltpu.async_copy(dstm_hbm.at[wid, j + 1], db, semI)

            @pl.when(j > 0)
            def _():
                pltpu.make_async_copy(buf_v, accum.at[da], semS).wait()

            @plsc.parallel_loop(0, B, unroll=2)
            def edge(e):
                As = Fs_v[e, pl.ds(HC, 16)]
                Gd = Gd_v[e, :]
                a1 = As + Gd
                a2 = maxs + Gd
                s1 = jnp.where(a1 >= 0, a1, 0.2 * a1)
                s2 = jnp.where(a2 >= 0, a2, 0.2 * a2)
                w = jnp.where(hmask, jnp.exp(s1 - s2), 0.0)
                buf_v[e, pl.ds(HC, 16)] = w
                for k in range(HC // 16):
                    head = (k * 16) // C
                    idx = jnp.full((16, 1), head, jnp.int32)
                    mult = lax.gather(w, idx, dnums, (1,),
                                      mode=lax.GatherScatterMode.PROMISE_IN_BOUNDS)
                    buf_v[e, pl.ds(k * 16, 16)] = Fs_v[e, pl.ds(k * 16, 16)] * mult

            pltpu.async_copy(buf_v, accum.at[da], semS, add=True)

            @pl.when(j + 1 < CH)
            def _():
                pltpu.make_async_copy(srcm_hbm.at[wid, 0], sb, semI).wait()
                pltpu.make_async_copy(dstm_hbm.at[wid, 0], db, semI).wait()
                pltpu.async_copy(F_hbm.at[sb], Fs_v, semF)
                pltpu.async_copy(G_hbm.at[db], Gd_v, semG)

        def pair(t, carry):
            j0 = 2 * t
            step(j0, src0, dst0, src1, dst1)
            step(j0 + 1, src1, dst1, src0, dst0)
            return carry

        lax.fori_loop(0, CH // 2, pair, 0)
        if CH % 2:
            step(CH - 1, src0, dst0, src1, dst1)
            pltpu.make_async_copy(buf_v, accum.at[dst0], semS).wait()
        else:
            pltpu.make_async_copy(buf_v, accum.at[dst1], semS).wait()

        plsc.subcore_barrier()
        pltpu.sync_copy(accum.at[pl.ds(row0, RPT)], out_hbm.at[c, pl.ds(row0, RPT)])

    return pl.kernel(
        body,
        out_type=jax.ShapeDtypeStruct((NC, NROWS, HCp), jnp.float32),
        mesh=mesh,
        scratch_types=[
            pltpu.VMEM((B,), jnp.int32),
            pltpu.VMEM((B,), jnp.int32),
            pltpu.VMEM((B,), jnp.int32),
            pltpu.VMEM((B,), jnp.int32),
            pltpu.VMEM((B, HCp), jnp.float32),
            pltpu.VMEM((B, 16), jnp.float32),
            pltpu.VMEM((B, HCp), jnp.float32),
            pltpu.VMEM((16,), jnp.float32),
            pltpu.VMEM_SHARED((NROWS, HCp), jnp.float32),
            pltpu.SemaphoreType.DMA,
            pltpu.SemaphoreType.DMA,
            pltpu.SemaphoreType.DMA,
            pltpu.SemaphoreType.DMA,
        ],
        compiler_params=pltpu.CompilerParams(use_tc_tiling_on_sc=False),
    )


def _tc1_body(x_ref, W_ref, Am_ref, Ad_ref, F_ref, G_ref, mx_ref):
    h = jnp.dot(x_ref[...], W_ref[...], preferred_element_type=jnp.float32)
    asrc = jnp.dot(h, Am_ref[...], preferred_element_type=jnp.float32)
    adst = jnp.dot(h, Ad_ref[...], preferred_element_type=jnp.float32)
    z = jnp.zeros_like(asrc)
    F_ref[:, :D] = h
    F_ref[:, D:D + 8] = asrc
    F_ref[:, D + 8:] = z
    G_ref[:, :8] = adst
    G_ref[:, 8:] = z
    bm = jnp.max(asrc, axis=0, keepdims=True)
    bm16 = jnp.concatenate([bm, jnp.zeros((1, 8), jnp.float32)], axis=1)
    pi = pl.program_id(0)

    @pl.when(pi == 0)
    def _():
        mx_ref[...] = bm16

    @pl.when(pi != 0)
    def _():
        mx_ref[...] = jnp.maximum(mx_ref[...], bm16)


def _elu(v):
    return jnp.where(v > 0, v, jnp.exp(jnp.minimum(v, 0.0)) - 1.0)


def _tc2_body(P_ref, b1_ref, W2_ref, S_ref, Am_ref, Ad_ref, F_ref, G_ref, mx_ref):
    p0 = P_ref[0]
    p1 = P_ref[1]
    num = p0[:, :D] + p1[:, :D]
    den8 = p0[:, D:D + 8] + p1[:, D:D + 8]
    den = jnp.dot(den8, S_ref[...], preferred_element_type=jnp.float32)
    x1 = _elu(num / (den + 1e-16) + b1_ref[...])
    h2 = jnp.dot(x1, W2_ref[...], preferred_element_type=jnp.float32)
    asrc16 = jnp.dot(h2, Am_ref[...], preferred_element_type=jnp.float32)
    adst16 = jnp.dot(h2, Ad_ref[...], preferred_element_type=jnp.float32)
    F_ref[:, :32] = h2
    F_ref[:, 32:] = asrc16
    G_ref[...] = adst16
    pi = pl.program_id(0)
    bm16 = jnp.max(asrc16, axis=0, keepdims=True)

    @pl.when(pi == 0)
    def _():
        mx_ref[...] = bm16

    @pl.when(pi != 0)
    def _():
        mx_ref[...] = jnp.maximum(mx_ref[...], bm16)


def _tc3_body(P_ref, b2_ref, Wh_ref, bh_ref, o_ref):
    p0 = P_ref[0]
    p1 = P_ref[1]
    num = p0[:, :32] + p1[:, :32]
    den = p0[:, 32:33] + p1[:, 32:33]
    x2 = _elu(num / (den + 1e-16) + b2_ref[...])
    o_ref[...] = jnp.dot(x2, Wh_ref[...], preferred_element_type=jnp.float32) + bh_ref[...]


def kernel(x, edge_index, W1, att_src1, att_dst1, b1, W2, att_src2, att_dst2, b2, Wh, bh):
    f32 = jnp.float32
    eye8 = jnp.eye(8, dtype=f32)
    Am1 = jnp.kron(eye8, jnp.ones((C1, 1), f32)) * att_src1.reshape(D, 1)
    Ad1 = jnp.kron(eye8, jnp.ones((C1, 1), f32)) * att_dst1.reshape(D, 1)
    S = jnp.kron(eye8, jnp.ones((1, C1), f32))
    Am2 = jnp.concatenate([att_src2.reshape(C2, 1), jnp.zeros((C2, 15), f32)], axis=1)
    Ad2 = jnp.concatenate([att_dst2.reshape(C2, 1), jnp.zeros((C2, 15), f32)], axis=1)

    loop = jnp.arange(N, dtype=jnp.int32)
    pad = E_PAD - EP
    srcm = jnp.concatenate([edge_index[0], loop, jnp.zeros((pad,), jnp.int32)]).reshape(NW, CH, B)
    dstm = jnp.concatenate([edge_index[1], loop, jnp.full((pad,), N, jnp.int32)]).reshape(NW, CH, B)

    F1, G1, mx1 = pl.pallas_call(
        _tc1_body,
        grid=(10,),
        in_specs=[
            pl.BlockSpec((1000, D), lambda i: (i, 0)),
            pl.BlockSpec((D, D), lambda i: (0, 0)),
            pl.BlockSpec((D, 8), lambda i: (0, 0)),
            pl.BlockSpec((D, 8), lambda i: (0, 0)),
        ],
        out_specs=[
            pl.BlockSpec((1000, D + 16), lambda i: (i, 0)),
            pl.BlockSpec((1000, 16), lambda i: (i, 0)),
            pl.BlockSpec((1, 16), lambda i: (0, 0)),
        ],
        out_shape=[
            jax.ShapeDtypeStruct((N, D + 16), f32),
            jax.ShapeDtypeStruct((N, 16), f32),
            jax.ShapeDtypeStruct((1, 16), f32),
        ],
    )(x, W1, Am1, Ad1)

    G1p = jnp.concatenate([G1, jnp.zeros((1, 16), f32)], axis=0)

    part1 = _edge_pass(D, H1, C1, N, N + 1)(F1, G1p, srcm, dstm, mx1.reshape(16))

    F2, G2, mx2 = pl.pallas_call(
        _tc2_body,
        grid=(10,),
        in_specs=[
            pl.BlockSpec((2, 1000, D + 16), lambda i: (0, i, 0)),
            pl.BlockSpec((1, D), lambda i: (0, 0)),
            pl.BlockSpec((D, 32), lambda i: (0, 0)),
            pl.BlockSpec((8, D), lambda i: (0, 0)),
            pl.BlockSpec((32, 16), lambda i: (0, 0)),
            pl.BlockSpec((32, 16), lambda i: (0, 0)),
        ],
        out_specs=[
            pl.BlockSpec((1000, 48), lambda i: (i, 0)),
            pl.BlockSpec((1000, 16), lambda i: (i, 0)),
            pl.BlockSpec((1, 16), lambda i: (0, 0)),
        ],
        out_shape=[
            jax.ShapeDtypeStruct((N, 48), f32),
            jax.ShapeDtypeStruct((N, 16), f32),
            jax.ShapeDtypeStruct((1, 16), f32),
        ],
    )(part1[:, :N], b1.reshape(1, D), W2, S, Am2, Ad2)

    G2p = jnp.concatenate([G2, jnp.zeros((1, 16), f32)], axis=0)

    part2 = _edge_pass(32, H2, C2, N, N + 1)(F2, G2p, srcm, dstm, mx2.reshape(16))

    out = pl.pallas_call(
        _tc3_body,
        grid=(10,),
        in_specs=[
            pl.BlockSpec((2, 1000, 48), lambda i: (0, i, 0)),
            pl.BlockSpec((1, 32), lambda i: (0, 0)),
            pl.BlockSpec((32, 1), lambda i: (0, 0)),
            pl.BlockSpec((1, 1), lambda i: (0, 0)),
        ],
        out_specs=pl.BlockSpec((1000, 1), lambda i: (i, 0)),
        out_shape=jax.ShapeDtypeStruct((N, 1), f32),
    )(part2[:, :N], b2.reshape(1, 32), Wh, bh.reshape(1, 1))

    return out


# speedup vs baseline: 103.8310x; 1.2103x over previous
import jax
import jax.numpy as jnp
from jax import lax
from jax.experimental import pallas as pl
from jax.experimental.pallas import tpu as pltpu
from jax.experimental.pallas import tpu_sc as plsc

N = 10000
E = 320000
D = 128
H1, C1 = 8, 16
H2, C2 = 1, 32

NROWS = 10016
NG = 10008
B = 128
NC, NS = 2, 16
NW = NC * NS
EP = E + N
CH = -(-EP // (NW * B))
E_PAD = NW * CH * B
RPT = NROWS // NS

_DNUMS = lax.GatherDimensionNumbers(
    offset_dims=(), collapsed_slice_dims=(0,), start_index_map=(0,))


def _lrelu(v):
    return jnp.where(v >= 0, v, 0.2 * v)


def _edge_compute(Fs_v, Gd_v, buf_v, maxs, hmask, HC, C):
    @plsc.parallel_loop(0, B, unroll=2)
    def edge(e):
        As = Fs_v[e, pl.ds(HC, 16)]
        Gd = Gd_v[e, :]
        w = jnp.where(hmask, jnp.exp(_lrelu(As + Gd) - _lrelu(maxs + Gd)), 0.0)
        buf_v[e, pl.ds(HC, 16)] = w
        for k in range(HC // 16):
            head = (k * 16) // C
            idx = jnp.full((16, 1), head, jnp.int32)
            mult = lax.gather(w, idx, _DNUMS, (1,),
                              mode=lax.GatherScatterMode.PROMISE_IN_BOUNDS)
            buf_v[e, pl.ds(k * 16, 16)] = Fs_v[e, pl.ds(k * 16, 16)] * mult


def _zero_accum(buf_v, accum, row0, HCp):
    zero16 = jnp.zeros((16,), jnp.float32)

    def zrow(r, carry):
        for k in range(HCp // 16):
            buf_v[r, pl.ds(k * 16, 16)] = zero16
        return carry

    lax.fori_loop(0, B, zrow, 0)
    off = 0
    while off < RPT:
        step = min(B, RPT - off)
        pltpu.sync_copy(buf_v.at[pl.ds(0, step)], accum.at[pl.ds(row0 + off, step)])
        off += step


def _edge_pass_l1(HC, H, C):
    HCp = HC + 16
    mesh = plsc.VectorSubcoreMesh(
        core_axis_name="c", subcore_axis_name="s", num_cores=NC, num_subcores=NS)

    def body(F_hbm, G_hbm, srcm_hbm, dstm_hbm, maxs_hbm, out_hbm,
             src0, dst0, src1, dst1, Fs_v, Gd_v, buf_v, maxs_v, accum,
             semF, semG, semI, semS):
        c = lax.axis_index("c")
        s = lax.axis_index("s")
        wid = s * NC + c
        row0 = s * RPT

        _zero_accum(buf_v, accum, row0, HCp)
        pltpu.sync_copy(maxs_hbm, maxs_v)
        maxs = maxs_v[...]
        hmask = lax.iota(jnp.int32, 16) < H

        plsc.subcore_barrier()

        pltpu.sync_copy(srcm_hbm.at[wid, 0], src0)
        pltpu.sync_copy(dstm_hbm.at[wid, 0], dst0)
        pltpu.async_copy(F_hbm.at[src0], Fs_v, semF)
        pltpu.async_copy(G_hbm.at[dst0], Gd_v, semG)

        def step(j, sa, da, sb, db):
            pltpu.make_async_copy(F_hbm.at[sa], Fs_v, semF).wait()
            pltpu.make_async_copy(G_hbm.at[da], Gd_v, semG).wait()

            @pl.when(j + 1 < CH)
            def _():
                pltpu.async_copy(srcm_hbm.at[wid, j + 1], sb, semI)
                pltpu.async_copy(dstm_hbm.at[wid, j + 1], db, semI)

            @pl.when(j > 0)
            def _():
                pltpu.make_async_copy(buf_v, accum.at[da], semS).wait()

            _edge_compute(Fs_v, Gd_v, buf_v, maxs, hmask, HC, C)
            pltpu.async_copy(buf_v, accum.at[da], semS, add=True)

            @pl.when(j + 1 < CH)
            def _():
                pltpu.make_async_copy(srcm_hbm.at[wid, 0], sb, semI).wait()
                pltpu.make_async_copy(dstm_hbm.at[wid, 0], db, semI).wait()
                pltpu.async_copy(F_hbm.at[sb], Fs_v, semF)
                pltpu.async_copy(G_hbm.at[db], Gd_v, semG)

        def pair(t, carry):
            j0 = 2 * t
            step(j0, src0, dst0, src1, dst1)
            step(j0 + 1, src1, dst1, src0, dst0)
            return carry

        lax.fori_loop(0, CH // 2, pair, 0)
        if CH % 2:
            step(CH - 1, src0, dst0, src1, dst1)
            pltpu.make_async_copy(buf_v, accum.at[dst0], semS).wait()
        else:
            pltpu.make_async_copy(buf_v, accum.at[dst1], semS).wait()

        plsc.subcore_barrier()
        pltpu.sync_copy(accum.at[pl.ds(row0, RPT)], out_hbm.at[c, pl.ds(row0, RPT)])

    return pl.kernel(
        body,
        out_type=jax.ShapeDtypeStruct((NC, NROWS, HCp), jnp.float32),
        mesh=mesh,
        scratch_types=[
            pltpu.VMEM((B,), jnp.int32),
            pltpu.VMEM((B,), jnp.int32),
            pltpu.VMEM((B,), jnp.int32),
            pltpu.VMEM((B,), jnp.int32),
            pltpu.VMEM((B, HCp), jnp.float32),
            pltpu.VMEM((B, 16), jnp.float32),
            pltpu.VMEM((B, HCp), jnp.float32),
            pltpu.VMEM((16,), jnp.float32),
            pltpu.VMEM_SHARED((NROWS, HCp), jnp.float32),
            pltpu.SemaphoreType.DMA,
            pltpu.SemaphoreType.DMA,
            pltpu.SemaphoreType.DMA,
            pltpu.SemaphoreType.DMA,
        ],
        compiler_params=pltpu.CompilerParams(use_tc_tiling_on_sc=False),
    )


def _edge_pass_l2(HC, H, C):
    HCp = HC + 16
    mesh = plsc.VectorSubcoreMesh(
        core_axis_name="c", subcore_axis_name="s", num_cores=NC, num_subcores=NS)

    def body(F_hbm, G_hbm, srcm_hbm, dstm_hbm, maxs_hbm, out_hbm,
             src_a, dst_a, Fs0, Gd0, buf0, Fs1, Gd1, buf1, maxs_v, accum,
             semF0, semG0, semS0, semF1, semG1, semS1):
        c = lax.axis_index("c")
        s = lax.axis_index("s")
        wid = s * NC + c
        row0 = s * RPT

        _zero_accum(buf0, accum, row0, HCp)
        pltpu.sync_copy(maxs_hbm, maxs_v)
        maxs = maxs_v[...]
        hmask = lax.iota(jnp.int32, 16) < H
        pltpu.sync_copy(srcm_hbm.at[wid], src_a)
        pltpu.sync_copy(dstm_hbm.at[wid], dst_a)

        plsc.subcore_barrier()

        bank0 = (Fs0, Gd0, buf0, semF0, semG0, semS0)
        bank1 = (Fs1, Gd1, buf1, semF1, semG1, semS1)

        pltpu.async_copy(F_hbm.at[src_a.at[0]], Fs0, semF0)
        pltpu.async_copy(G_hbm.at[dst_a.at[0]], Gd0, semG0)

        def step(j, bank, nbank):
            Fs, Gd, buf, semF, semG, semS = bank
            nFs, nGd, nbuf, nsemF, nsemG, nsemS = nbank

            @pl.when(j + 1 < CH)
            def _():
                pltpu.async_copy(F_hbm.at[src_a.at[j + 1]], nFs, nsemF)
                pltpu.async_copy(G_hbm.at[dst_a.at[j + 1]], nGd, nsemG)

            pltpu.make_async_copy(F_hbm.at[src_a.at[j]], Fs, semF).wait()
            pltpu.make_async_copy(G_hbm.at[dst_a.at[j]], Gd, semG).wait()

            @pl.when(j > 1)
            def _():
                pltpu.make_async_copy(buf, accum.at[dst_a.at[j]], semS).wait()

            _edge_compute(Fs, Gd, buf, maxs, hmask, HC, C)
            pltpu.async_copy(buf, accum.at[dst_a.at[j]], semS, add=True)

        def pair(t, carry):
            j0 = 2 * t
            step(j0, bank0, bank1)
            step(j0 + 1, bank1, bank0)
            return carry

        lax.fori_loop(0, CH // 2, pair, 0)
        if CH % 2:
            step(CH - 1, bank0, bank1)
            pltpu.make_async_copy(buf1, accum.at[dst_a.at[0]], semS1).wait()
            pltpu.make_async_copy(buf0, accum.at[dst_a.at[0]], semS0).wait()
        else:
            pltpu.make_async_copy(buf0, accum.at[dst_a.at[0]], semS0).wait()
            pltpu.make_async_copy(buf1, accum.at[dst_a.at[0]], semS1).wait()

        plsc.subcore_barrier()
        pltpu.sync_copy(accum.at[pl.ds(row0, RPT)], out_hbm.at[c, pl.ds(row0, RPT)])

    return pl.kernel(
        body,
        out_type=jax.ShapeDtypeStruct((NC, NROWS, HCp), jnp.float32),
        mesh=mesh,
        scratch_types=[
            pltpu.VMEM((CH, B), jnp.int32),
            pltpu.VMEM((CH, B), jnp.int32),
            pltpu.VMEM((B, HCp), jnp.float32),
            pltpu.VMEM((B, 16), jnp.float32),
            pltpu.VMEM((B, HCp), jnp.float32),
            pltpu.VMEM((B, HCp), jnp.float32),
            pltpu.VMEM((B, 16), jnp.float32),
            pltpu.VMEM((B, HCp), jnp.float32),
            pltpu.VMEM((16,), jnp.float32),
            pltpu.VMEM_SHARED((NROWS, HCp), jnp.float32),
            pltpu.SemaphoreType.DMA,
            pltpu.SemaphoreType.DMA,
            pltpu.SemaphoreType.DMA,
            pltpu.SemaphoreType.DMA,
            pltpu.SemaphoreType.DMA,
            pltpu.SemaphoreType.DMA,
        ],
        compiler_params=pltpu.CompilerParams(use_tc_tiling_on_sc=False),
    )


def _tc1_body(x_ref, W_ref, Am_ref, Ad_ref, F_ref, G_ref, mx_ref):
    h = jnp.dot(x_ref[...], W_ref[...], preferred_element_type=jnp.float32)
    asrc = jnp.dot(h, Am_ref[...], preferred_element_type=jnp.float32)
    adst = jnp.dot(h, Ad_ref[...], preferred_element_type=jnp.float32)
    z = jnp.zeros_like(asrc)
    F_ref[:, :D] = h
    F_ref[:, D:D + 8] = asrc
    F_ref[:, D + 8:] = z
    G_ref[...] = jnp.zeros((NG, 16), jnp.float32)
    G_ref[:N, :8] = adst
    bm = jnp.max(asrc, axis=0, keepdims=True)
    mx_ref[...] = jnp.concatenate([bm, jnp.zeros((1, 8), jnp.float32)], axis=1)


def _elu(v):
    return jnp.where(v > 0, v, jnp.exp(jnp.minimum(v, 0.0)) - 1.0)


def _tc2_body(P_ref, b1_ref, W2_ref, S_ref, Am_ref, Ad_ref, F_ref, G_ref, mx_ref):
    p0 = P_ref[0, :N]
    p1 = P_ref[1, :N]
    num = p0[:, :D] + p1[:, :D]
    den8 = p0[:, D:D + 8] + p1[:, D:D + 8]
    den = jnp.dot(den8, S_ref[...], preferred_element_type=jnp.float32)
    x1 = _elu(num / (den + 1e-16) + b1_ref[...])
    h2 = jnp.dot(x1, W2_ref[...], preferred_element_type=jnp.float32)
    asrc16 = jnp.dot(h2, Am_ref[...], preferred_element_type=jnp.float32)
    adst16 = jnp.dot(h2, Ad_ref[...], preferred_element_type=jnp.float32)
    F_ref[:, :32] = h2
    F_ref[:, 32:] = asrc16
    G_ref[...] = jnp.zeros((NG, 16), jnp.float32)
    G_ref[:N, :] = adst16
    mx_ref[...] = jnp.max(asrc16, axis=0, keepdims=True)


def _tc3_body(P_ref, b2_ref, Wh_ref, bh_ref, o_ref):
    p0 = P_ref[0, :N]
    p1 = P_ref[1, :N]
    num = p0[:, :32] + p1[:, :32]
    den = p0[:, 32:33] + p1[:, 32:33]
    x2 = _elu(num / (den + 1e-16) + b2_ref[...])
    o_ref[...] = jnp.dot(x2, Wh_ref[...], preferred_element_type=jnp.float32) + bh_ref[...]


def kernel(x, edge_index, W1, att_src1, att_dst1, b1, W2, att_src2, att_dst2, b2, Wh, bh):
    f32 = jnp.float32
    eye8 = jnp.eye(8, dtype=f32)
    Am1 = jnp.kron(eye8, jnp.ones((C1, 1), f32)) * att_src1.reshape(D, 1)
    Ad1 = jnp.kron(eye8, jnp.ones((C1, 1), f32)) * att_dst1.reshape(D, 1)
    S = jnp.kron(eye8, jnp.ones((1, C1), f32))
    Am2 = jnp.concatenate([att_src2.reshape(C2, 1), jnp.zeros((C2, 15), f32)], axis=1)
    Ad2 = jnp.concatenate([att_dst2.reshape(C2, 1), jnp.zeros((C2, 15), f32)], axis=1)

    loop = jnp.arange(N, dtype=jnp.int32)
    pad = E_PAD - EP
    srcm = jnp.concatenate([edge_index[0], loop, jnp.zeros((pad,), jnp.int32)]).reshape(NW, CH, B)
    dstm = jnp.concatenate([edge_index[1], loop, jnp.full((pad,), N, jnp.int32)]).reshape(NW, CH, B)

    F1, G1, mx1 = pl.pallas_call(
        _tc1_body,
        grid=(1,),
        in_specs=[
            pl.BlockSpec((N, D), lambda i: (0, 0)),
            pl.BlockSpec((D, D), lambda i: (0, 0)),
            pl.BlockSpec((D, 8), lambda i: (0, 0)),
            pl.BlockSpec((D, 8), lambda i: (0, 0)),
        ],
        out_specs=[
            pl.BlockSpec((N, D + 16), lambda i: (0, 0)),
            pl.BlockSpec((NG, 16), lambda i: (0, 0)),
            pl.BlockSpec((1, 16), lambda i: (0, 0)),
        ],
        out_shape=[
            jax.ShapeDtypeStruct((N, D + 16), f32),
            jax.ShapeDtypeStruct((NG, 16), f32),
            jax.ShapeDtypeStruct((1, 16), f32),
        ],
    )(x, W1, Am1, Ad1)

    part1 = _edge_pass_l1(D, H1, C1)(F1, G1, srcm, dstm, mx1.reshape(16))

    F2, G2, mx2 = pl.pallas_call(
        _tc2_body,
        grid=(1,),
        in_specs=[
            pl.BlockSpec((2, NROWS, D + 16), lambda i: (0, 0, 0)),
            pl.BlockSpec((1, D), lambda i: (0, 0)),
            pl.BlockSpec((D, 32), lambda i: (0, 0)),
            pl.BlockSpec((8, D), lambda i: (0, 0)),
            pl.BlockSpec((32, 16), lambda i: (0, 0)),
            pl.BlockSpec((32, 16), lambda i: (0, 0)),
        ],
        out_specs=[
            pl.BlockSpec((N, 48), lambda i: (0, 0)),
            pl.BlockSpec((NG, 16), lambda i: (0, 0)),
            pl.BlockSpec((1, 16), lambda i: (0, 0)),
        ],
        out_shape=[
            jax.ShapeDtypeStruct((N, 48), f32),
            jax.ShapeDtypeStruct((NG, 16), f32),
            jax.ShapeDtypeStruct((1, 16), f32),
        ],
    )(part1, b1.reshape(1, D), W2, S, Am2, Ad2)

    part2 = _edge_pass_l2(32, H2, C2)(F2, G2, srcm, dstm, mx2.reshape(16))

    out = pl.pallas_call(
        _tc3_body,
        grid=(1,),
        in_specs=[
            pl.BlockSpec((2, NROWS, 48), lambda i: (0, 0, 0)),
            pl.BlockSpec((1, 32), lambda i: (0, 0)),
            pl.BlockSpec((32, 1), lambda i: (0, 0)),
            pl.BlockSpec((1, 1), lambda i: (0, 0)),
        ],
        out_specs=pl.BlockSpec((N, 1), lambda i: (0, 0)),
        out_shape=jax.ShapeDtypeStruct((N, 1), f32),
    )(part2, b2.reshape(1, 32), Wh, bh.reshape(1, 1))

    return out


# speedup vs baseline: 117.0138x; 1.1270x over previous
import jax
import jax.numpy as jnp
from jax import lax
from jax.experimental import pallas as pl
from jax.experimental.pallas import tpu as pltpu
from jax.experimental.pallas import tpu_sc as plsc

N = 10000
E = 320000
D = 128
H1, C1 = 8, 16
H2, C2 = 1, 32

NROWS = 10016
NG = 10008
B = 128
NC, NS = 2, 16
NW = NC * NS
EP = E + N
CH = -(-EP // (NW * B))
E_PAD = NW * CH * B
RPT = NROWS // NS

_DNUMS = lax.GatherDimensionNumbers(
    offset_dims=(), collapsed_slice_dims=(0,), start_index_map=(0,))


def _lrelu(v):
    return jnp.where(v >= 0, v, 0.2 * v)


def _edge_compute(Fs_v, Gd_v, buf_v, maxs, hmask, HC, C):
    nB = buf_v.shape[0]

    @plsc.parallel_loop(0, nB, unroll=2)
    def edge(e):
        As = Fs_v[e, pl.ds(HC, 16)]
        Gd = Gd_v[e, :]
        w = jnp.where(hmask, jnp.exp(_lrelu(As + Gd) - _lrelu(maxs + Gd)), 0.0)
        buf_v[e, pl.ds(HC, 16)] = w
        for k in range(HC // 16):
            head = (k * 16) // C
            idx = jnp.full((16, 1), head, jnp.int32)
            mult = lax.gather(w, idx, _DNUMS, (1,),
                              mode=lax.GatherScatterMode.PROMISE_IN_BOUNDS)
            buf_v[e, pl.ds(k * 16, 16)] = Fs_v[e, pl.ds(k * 16, 16)] * mult


def _zero_accum(buf_v, accum, row0, HCp):
    zero16 = jnp.zeros((16,), jnp.float32)

    def zrow(r, carry):
        for k in range(HCp // 16):
            buf_v[r, pl.ds(k * 16, 16)] = zero16
        return carry

    nB = buf_v.shape[0]
    lax.fori_loop(0, nB, zrow, 0)
    off = 0
    while off < RPT:
        step = min(nB, RPT - off)
        pltpu.sync_copy(buf_v.at[pl.ds(0, step)], accum.at[pl.ds(row0 + off, step)])
        off += step


def _edge_pass_l1(HC, H, C):
    HCp = HC + 16
    B1, CH1 = 64, 2 * CH
    mesh = plsc.VectorSubcoreMesh(
        core_axis_name="c", subcore_axis_name="s", num_cores=NC, num_subcores=NS)

    def body(F_hbm, G_hbm, srcm_hbm, dstm_hbm, maxs_hbm, out_hbm,
             src0, dg0, ds0, src1, dg1, ds1, Fs0, Gd0, buf0, Fs1, Gd1, buf1,
             maxs_v, accum,
             semF0, semG0, semF1, semG1, semIg0, semIg1, semIs0, semIs1, semS):
        c = lax.axis_index("c")
        s = lax.axis_index("s")
        wid = s * NC + c
        row0 = s * RPT

        _zero_accum(buf0, accum, row0, HCp)
        pltpu.sync_copy(maxs_hbm, maxs_v)
        maxs = maxs_v[...]
        hmask = lax.iota(jnp.int32, 16) < H

        plsc.subcore_barrier()

        bank0 = (src0, dg0, ds0, Fs0, Gd0, buf0, semF0, semG0, semIg0, semIs0)
        bank1 = (src1, dg1, ds1, Fs1, Gd1, buf1, semF1, semG1, semIg1, semIs1)

        pltpu.sync_copy(srcm_hbm.at[wid, 0], src0)
        pltpu.sync_copy(dstm_hbm.at[wid, 0], dg0)
        pltpu.sync_copy(dstm_hbm.at[wid, 0], ds0)
        pltpu.sync_copy(srcm_hbm.at[wid, 1], src1)
        pltpu.sync_copy(dstm_hbm.at[wid, 1], dg1)
        pltpu.async_copy(F_hbm.at[src0], Fs0, semF0)
        pltpu.async_copy(G_hbm.at[dg0], Gd0, semG0)

        def step(j, bank, nbank):
            srcA, dgA, dsA, Fs, Gd, buf, semF, semG, semIg, semIs = bank
            srcB, dgB, dsB, nFs, nGd, nbuf, nsemF, nsemG, nsemIg, nsemIs = nbank

            @pl.when(j + 1 < CH1)
            def _():
                @pl.when(j > 0)
                def _():
                    pltpu.make_async_copy(srcm_hbm.at[wid, 0], srcB, nsemIg).wait()
                    pltpu.make_async_copy(dstm_hbm.at[wid, 0], dgB, nsemIg).wait()
                pltpu.async_copy(F_hbm.at[srcB], nFs, nsemF)
                pltpu.async_copy(G_hbm.at[dgB], nGd, nsemG)

            pltpu.make_async_copy(F_hbm.at[srcA], Fs, semF).wait()
            pltpu.make_async_copy(G_hbm.at[dgA], Gd, semG).wait()

            @pl.when(j + 2 < CH1)
            def _():
                pltpu.async_copy(srcm_hbm.at[wid, j + 2], srcA, semIg)
                pltpu.async_copy(dstm_hbm.at[wid, j + 2], dgA, semIg)

            _edge_compute(Fs, Gd, buf, maxs, hmask, HC, C)

            @pl.when(j > 0)
            def _():
                pltpu.make_async_copy(nbuf, accum.at[dsB], semS).wait()

            @pl.when(j + 1 < CH1)
            def _():
                pltpu.async_copy(dstm_hbm.at[wid, j + 1], dsB, nsemIs)

            @pl.when(j > 0)
            def _():
                pltpu.make_async_copy(dstm_hbm.at[wid, 0], dsA, semIs).wait()
            pltpu.async_copy(buf, accum.at[dsA], semS, add=True)

        def pair(t, carry):
            j0 = 2 * t
            step(j0, bank0, bank1)
            step(j0 + 1, bank1, bank0)
            return carry

        lax.fori_loop(0, CH1 // 2, pair, 0)
        if CH1 % 2:
            step(CH1 - 1, bank0, bank1)
            pltpu.make_async_copy(buf0, accum.at[ds0], semS).wait()
        else:
            pltpu.make_async_copy(buf1, accum.at[ds1], semS).wait()

        plsc.subcore_barrier()
        pltpu.sync_copy(accum.at[pl.ds(row0, RPT)], out_hbm.at[c, pl.ds(row0, RPT)])

    return pl.kernel(
        body,
        out_type=jax.ShapeDtypeStruct((NC, NROWS, HCp), jnp.float32),
        mesh=mesh,
        scratch_types=[
            pltpu.VMEM((B1,), jnp.int32),
            pltpu.VMEM((B1,), jnp.int32),
            pltpu.VMEM((B1,), jnp.int32),
            pltpu.VMEM((B1,), jnp.int32),
            pltpu.VMEM((B1,), jnp.int32),
            pltpu.VMEM((B1,), jnp.int32),
            pltpu.VMEM((B1, HCp), jnp.float32),
            pltpu.VMEM((B1, 16), jnp.float32),
            pltpu.VMEM((B1, HCp), jnp.float32),
            pltpu.VMEM((B1, HCp), jnp.float32),
            pltpu.VMEM((B1, 16), jnp.float32),
            pltpu.VMEM((B1, HCp), jnp.float32),
            pltpu.VMEM((16,), jnp.float32),
            pltpu.VMEM_SHARED((NROWS, HCp), jnp.float32),
            pltpu.SemaphoreType.DMA,
            pltpu.SemaphoreType.DMA,
            pltpu.SemaphoreType.DMA,
            pltpu.SemaphoreType.DMA,
            pltpu.SemaphoreType.DMA,
            pltpu.SemaphoreType.DMA,
            pltpu.SemaphoreType.DMA,
            pltpu.SemaphoreType.DMA,
            pltpu.SemaphoreType.DMA,
        ],
        compiler_params=pltpu.CompilerParams(use_tc_tiling_on_sc=False),
    )


def _edge_pass_l2(HC, H, C):
    HCp = HC + 16
    mesh = plsc.VectorSubcoreMesh(
        core_axis_name="c", subcore_axis_name="s", num_cores=NC, num_subcores=NS)

    def body(F_hbm, G_hbm, srcm_hbm, dstm_hbm, maxs_hbm, out_hbm,
             src_a, dst_a, Fs0, Gd0, buf0, Fs1, Gd1, buf1, maxs_v, accum,
             semF0, semG0, semS0, semF1, semG1, semS1):
        c = lax.axis_index("c")
        s = lax.axis_index("s")
        wid = s * NC + c
        row0 = s * RPT

        _zero_accum(buf0, accum, row0, HCp)
        pltpu.sync_copy(maxs_hbm, maxs_v)
        maxs = maxs_v[...]
        hmask = lax.iota(jnp.int32, 16) < H
        pltpu.sync_copy(srcm_hbm.at[wid], src_a)
        pltpu.sync_copy(dstm_hbm.at[wid], dst_a)

        plsc.subcore_barrier()

        bank0 = (Fs0, Gd0, buf0, semF0, semG0, semS0)
        bank1 = (Fs1, Gd1, buf1, semF1, semG1, semS1)

        pltpu.async_copy(F_hbm.at[src_a.at[0]], Fs0, semF0)
        pltpu.async_copy(G_hbm.at[dst_a.at[0]], Gd0, semG0)

        def step(j, bank, nbank):
            Fs, Gd, buf, semF, semG, semS = bank
            nFs, nGd, nbuf, nsemF, nsemG, nsemS = nbank

            @pl.when(j + 1 < CH)
            def _():
                pltpu.async_copy(F_hbm.at[src_a.at[j + 1]], nFs, nsemF)
                pltpu.async_copy(G_hbm.at[dst_a.at[j + 1]], nGd, nsemG)

            pltpu.make_async_copy(F_hbm.at[src_a.at[j]], Fs, semF).wait()
            pltpu.make_async_copy(G_hbm.at[dst_a.at[j]], Gd, semG).wait()

            @pl.when(j > 1)
            def _():
                pltpu.make_async_copy(buf, accum.at[dst_a.at[j]], semS).wait()

            _edge_compute(Fs, Gd, buf, maxs, hmask, HC, C)
            pltpu.async_copy(buf, accum.at[dst_a.at[j]], semS, add=True)

        def pair(t, carry):
            j0 = 2 * t
            step(j0, bank0, bank1)
            step(j0 + 1, bank1, bank0)
            return carry

        lax.fori_loop(0, CH // 2, pair, 0)
        if CH % 2:
            step(CH - 1, bank0, bank1)
            pltpu.make_async_copy(buf1, accum.at[dst_a.at[0]], semS1).wait()
            pltpu.make_async_copy(buf0, accum.at[dst_a.at[0]], semS0).wait()
        else:
            pltpu.make_async_copy(buf0, accum.at[dst_a.at[0]], semS0).wait()
            pltpu.make_async_copy(buf1, accum.at[dst_a.at[0]], semS1).wait()

        plsc.subcore_barrier()
        pltpu.sync_copy(accum.at[pl.ds(row0, RPT)], out_hbm.at[c, pl.ds(row0, RPT)])

    return pl.kernel(
        body,
        out_type=jax.ShapeDtypeStruct((NC, NROWS, HCp), jnp.float32),
        mesh=mesh,
        scratch_types=[
            pltpu.VMEM((CH, B), jnp.int32),
            pltpu.VMEM((CH, B), jnp.int32),
            pltpu.VMEM((B, HCp), jnp.float32),
            pltpu.VMEM((B, 16), jnp.float32),
            pltpu.VMEM((B, HCp), jnp.float32),
            pltpu.VMEM((B, HCp), jnp.float32),
            pltpu.VMEM((B, 16), jnp.float32),
            pltpu.VMEM((B, HCp), jnp.float32),
            pltpu.VMEM((16,), jnp.float32),
            pltpu.VMEM_SHARED((NROWS, HCp), jnp.float32),
            pltpu.SemaphoreType.DMA,
            pltpu.SemaphoreType.DMA,
            pltpu.SemaphoreType.DMA,
            pltpu.SemaphoreType.DMA,
            pltpu.SemaphoreType.DMA,
            pltpu.SemaphoreType.DMA,
        ],
        compiler_params=pltpu.CompilerParams(use_tc_tiling_on_sc=False),
    )


def _tc1_body(x_ref, W_ref, Am_ref, Ad_ref, F_ref, G_ref, mx_ref):
    h = jnp.dot(x_ref[...], W_ref[...], preferred_element_type=jnp.float32)
    asrc = jnp.dot(h, Am_ref[...], preferred_element_type=jnp.float32)
    adst = jnp.dot(h, Ad_ref[...], preferred_element_type=jnp.float32)
    z = jnp.zeros_like(asrc)
    F_ref[:, :D] = h
    F_ref[:, D:D + 8] = asrc
    F_ref[:, D + 8:] = z
    G_ref[...] = jnp.zeros((NG, 16), jnp.float32)
    G_ref[:N, :8] = adst
    bm = jnp.max(asrc, axis=0, keepdims=True)
    mx_ref[...] = jnp.concatenate([bm, jnp.zeros((1, 8), jnp.float32)], axis=1)


def _elu(v):
    return jnp.where(v > 0, v, jnp.exp(jnp.minimum(v, 0.0)) - 1.0)


def _tc2_body(P_ref, b1_ref, W2_ref, S_ref, Am_ref, Ad_ref, F_ref, G_ref, mx_ref):
    p0 = P_ref[0, :N]
    p1 = P_ref[1, :N]
    num = p0[:, :D] + p1[:, :D]
    den8 = p0[:, D:D + 8] + p1[:, D:D + 8]
    den = jnp.dot(den8, S_ref[...], preferred_element_type=jnp.float32)
    x1 = _elu(num / (den + 1e-16) + b1_ref[...])
    h2 = jnp.dot(x1, W2_ref[...], preferred_element_type=jnp.float32)
    asrc16 = jnp.dot(h2, Am_ref[...], preferred_element_type=jnp.float32)
    adst16 = jnp.dot(h2, Ad_ref[...], preferred_element_type=jnp.float32)
    F_ref[:, :32] = h2
    F_ref[:, 32:] = asrc16
    G_ref[...] = jnp.zeros((NG, 16), jnp.float32)
    G_ref[:N, :] = adst16
    mx_ref[...] = jnp.max(asrc16, axis=0, keepdims=True)


def _tc3_body(P_ref, b2_ref, Wh_ref, bh_ref, o_ref):
    p0 = P_ref[0, :N]
    p1 = P_ref[1, :N]
    num = p0[:, :32] + p1[:, :32]
    den = p0[:, 32:33] + p1[:, 32:33]
    x2 = _elu(num / (den + 1e-16) + b2_ref[...])
    o_ref[...] = jnp.dot(x2, Wh_ref[...], preferred_element_type=jnp.float32) + bh_ref[...]


def kernel(x, edge_index, W1, att_src1, att_dst1, b1, W2, att_src2, att_dst2, b2, Wh, bh):
    f32 = jnp.float32
    eye8 = jnp.eye(8, dtype=f32)
    Am1 = jnp.kron(eye8, jnp.ones((C1, 1), f32)) * att_src1.reshape(D, 1)
    Ad1 = jnp.kron(eye8, jnp.ones((C1, 1), f32)) * att_dst1.reshape(D, 1)
    S = jnp.kron(eye8, jnp.ones((1, C1), f32))
    Am2 = jnp.concatenate([att_src2.reshape(C2, 1), jnp.zeros((C2, 15), f32)], axis=1)
    Ad2 = jnp.concatenate([att_dst2.reshape(C2, 1), jnp.zeros((C2, 15), f32)], axis=1)

    loop = jnp.arange(N, dtype=jnp.int32)
    pad = E_PAD - EP
    pad_dst = N + (jnp.arange(pad, dtype=jnp.int32) % 8)
    srcm = jnp.concatenate([edge_index[0], loop, jnp.zeros((pad,), jnp.int32)]).reshape(NW, CH, B)
    dstm = jnp.concatenate([edge_index[1], loop, pad_dst]).reshape(NW, CH, B)

    F1, G1, mx1 = pl.pallas_call(
        _tc1_body,
        grid=(1,),
        in_specs=[
            pl.BlockSpec((N, D), lambda i: (0, 0)),
            pl.BlockSpec((D, D), lambda i: (0, 0)),
            pl.BlockSpec((D, 8), lambda i: (0, 0)),
            pl.BlockSpec((D, 8), lambda i: (0, 0)),
        ],
        out_specs=[
            pl.BlockSpec((N, D + 16), lambda i: (0, 0)),
            pl.BlockSpec((NG, 16), lambda i: (0, 0)),
            pl.BlockSpec((1, 16), lambda i: (0, 0)),
        ],
        out_shape=[
            jax.ShapeDtypeStruct((N, D + 16), f32),
            jax.ShapeDtypeStruct((NG, 16), f32),
            jax.ShapeDtypeStruct((1, 16), f32),
        ],
    )(x, W1, Am1, Ad1)

    part1 = _edge_pass_l1(D, H1, C1)(
        F1, G1, srcm.reshape(NW, 2 * CH, 64), dstm.reshape(NW, 2 * CH, 64),
        mx1.reshape(16))

    F2, G2, mx2 = pl.pallas_call(
        _tc2_body,
        grid=(1,),
        in_specs=[
            pl.BlockSpec((2, NROWS, D + 16), lambda i: (0, 0, 0)),
            pl.BlockSpec((1, D), lambda i: (0, 0)),
            pl.BlockSpec((D, 32), lambda i: (0, 0)),
            pl.BlockSpec((8, D), lambda i: (0, 0)),
            pl.BlockSpec((32, 16), lambda i: (0, 0)),
            pl.BlockSpec((32, 16), lambda i: (0, 0)),
        ],
        out_specs=[
            pl.BlockSpec((N, 48), lambda i: (0, 0)),
            pl.BlockSpec((NG, 16), lambda i: (0, 0)),
            pl.BlockSpec((1, 16), lambda i: (0, 0)),
        ],
        out_shape=[
            jax.ShapeDtypeStruct((N, 48), f32),
            jax.ShapeDtypeStruct((NG, 16), f32),
            jax.ShapeDtypeStruct((1, 16), f32),
        ],
    )(part1, b1.reshape(1, D), W2, S, Am2, Ad2)

    part2 = _edge_pass_l2(32, H2, C2)(F2, G2, srcm, dstm, mx2.reshape(16))

    out = pl.pallas_call(
        _tc3_body,
        grid=(1,),
        in_specs=[
            pl.BlockSpec((2, NROWS, 48), lambda i: (0, 0, 0)),
            pl.BlockSpec((1, 32), lambda i: (0, 0)),
            pl.BlockSpec((32, 1), lambda i: (0, 0)),
            pl.BlockSpec((1, 1), lambda i: (0, 0)),
        ],
        out_specs=pl.BlockSpec((N, 1), lambda i: (0, 0)),
        out_shape=jax.ShapeDtypeStruct((N, 1), f32),
    )(part2, b2.reshape(1, 32), Wh, bh.reshape(1, 1))

    return out
